# baseline (speedup 1.0000x reference)
import functools

import jax
import jax.numpy as jnp
from jax import lax
from jax.experimental import pallas as pl
from jax.experimental.pallas import tpu as pltpu
from jax.experimental.pallas import tpu_sc as plsc

N_NODES = 10000
H = 128
OUT = 3
L = 4

NC = 2
NS = 16
NW = NC * NS
CHUNK = 128
CPW = 80
TCH = NW * CPW
CPW0 = 124
CPW1 = 36
EPAD = TCH * CHUNK
NACC = 10240
ZROWS = NACC // NS
BR = 1000


def _edge_accum_body(t_hbm, e_hbm, zeros_hbm, out0_hbm, out1_hbm,
                     eb0, eb1, rows0, rows1, shared,
                     gsem0, gsem1, ssem0, ssem1):
    c = lax.axis_index("c")
    s = lax.axis_index("s")
    base = jnp.where(c == 0, s * CPW0, 16 * CPW0 + s * CPW1)
    nch = jnp.where(c == 0, CPW0, CPW1)
    pltpu.sync_copy(zeros_hbm.at[pl.ds(s * ZROWS, ZROWS)],
                    shared.at[pl.ds(s * ZROWS, ZROWS)])
    plsc.subcore_barrier()

    eb = (eb0, eb1)
    rows = (rows0, rows1)
    gsem = (gsem0, gsem1)
    ssem = (ssem0, ssem1)

    def drain(sem):
        pltpu.make_async_copy(zeros_hbm.at[pl.ds(0, CHUNK)], rows[0],
                              sem).wait()

    pltpu.sync_copy(e_hbm.at[base], eb[0])
    pltpu.async_copy(t_hbm.at[eb[0].at[0]], rows[0], gsem[0])

    def body(j2, carry):
        for phase in range(2):
            j = 2 * j2 + phase
            cur, nxt = phase, 1 - phase

            if phase == 0:
                @pl.when(j2 > 0)
                def _():
                    drain(ssem[nxt])
                pltpu.sync_copy(e_hbm.at[base + j + 1], eb[nxt])
                pltpu.async_copy(t_hbm.at[eb[nxt].at[0]], rows[nxt],
                                 gsem[nxt])
            else:
                @pl.when(j + 1 < nch)
                def _():
                    drain(ssem[nxt])
                    pltpu.sync_copy(e_hbm.at[base + j + 1], eb[nxt])
                    pltpu.async_copy(t_hbm.at[eb[nxt].at[0]], rows[nxt],
                                     gsem[nxt])

            drain(gsem[cur])
            pltpu.async_copy(rows[cur], shared.at[eb[cur].at[1]], ssem[cur],
                             add=True)
        return carry

    lax.fori_loop(0, nch // 2, body, 0)
    drain(ssem[0])
    drain(ssem[1])
    plsc.subcore_barrier()
    @pl.when(c == 0)
    def _():
        pltpu.sync_copy(shared.at[pl.ds(s * ZROWS, ZROWS)],
                        out0_hbm.at[pl.ds(s * ZROWS, ZROWS)])
    @pl.when(c == 1)
    def _():
        pltpu.sync_copy(shared.at[pl.ds(s * ZROWS, ZROWS)],
                        out1_hbm.at[pl.ds(s * ZROWS, ZROWS)])


def _deg_accum_body(e_hbm, ones_hbm, zeros_hbm, out0_hbm, out1_hbm,
                    ev, ones_v, shared):
    c = lax.axis_index("c")
    s = lax.axis_index("s")
    w = s * NC + c
    pltpu.sync_copy(ones_hbm, ones_v)
    pltpu.sync_copy(zeros_hbm.at[pl.ds(s * ZROWS, ZROWS)],
                    shared.at[pl.ds(s * ZROWS, ZROWS)])
    plsc.subcore_barrier()

    def body(j, carry):
        pltpu.sync_copy(e_hbm.at[w * CPW + j], ev)
        pltpu.sync_copy(ones_v, shared.at[ev.at[1]], add=True)
        return carry

    lax.fori_loop(0, CPW, body, 0)
    plsc.subcore_barrier()
    @pl.when(c == 0)
    def _():
        pltpu.sync_copy(shared.at[pl.ds(s * ZROWS, ZROWS)],
                        out0_hbm.at[pl.ds(s * ZROWS, ZROWS)])
    @pl.when(c == 1)
    def _():
        pltpu.sync_copy(shared.at[pl.ds(s * ZROWS, ZROWS)],
                        out1_hbm.at[pl.ds(s * ZROWS, ZROWS)])


@functools.cache
def _sc_kernels():
    mesh = plsc.VectorSubcoreMesh(core_axis_name="c", subcore_axis_name="s")
    part = jax.ShapeDtypeStruct((NACC, H), jnp.float32)
    edge_accum = pl.kernel(
        _edge_accum_body,
        mesh=mesh,
        out_type=(part, part),
        scratch_types=[
            pltpu.VMEM((2, CHUNK), jnp.int32),
            pltpu.VMEM((2, CHUNK), jnp.int32),
            pltpu.VMEM((CHUNK, H), jnp.float32),
            pltpu.VMEM((CHUNK, H), jnp.float32),
            pltpu.VMEM_SHARED((NACC, H), jnp.float32),
            pltpu.SemaphoreType.DMA,
            pltpu.SemaphoreType.DMA,
            pltpu.SemaphoreType.DMA,
            pltpu.SemaphoreType.DMA,
        ],
    )
    deg_accum = pl.kernel(
        _deg_accum_body,
        mesh=mesh,
        out_type=(part, part),
        scratch_types=[
            pltpu.VMEM((2, CHUNK), jnp.int32),
            pltpu.VMEM((CHUNK, H), jnp.float32),
            pltpu.VMEM_SHARED((NACC, H), jnp.float32),
        ],
    )
    return edge_accum, deg_accum


def _enc_body(x_ref, mean_ref, std_ref, w1_ref, b1_ref, w2_ref, b2_ref,
              g_ref, beta_ref, o_ref):
    xn = (x_ref[...] - mean_ref[...]) / std_ref[...]
    a = jnp.maximum(jnp.dot(xn, w1_ref[...],
                            preferred_element_type=jnp.float32)
                    + b1_ref[...], 0.0)
    h = jnp.dot(a, w2_ref[...], preferred_element_type=jnp.float32) + b2_ref[...]
    mu = jnp.mean(h, axis=-1, keepdims=True)
    var = jnp.mean((h - mu) * (h - mu), axis=-1, keepdims=True)
    o_ref[...] = (h - mu) * lax.rsqrt(var + 1e-5) * g_ref[...] + beta_ref[...]


def _dinv_body(d0_ref, d1_ref, o_ref):
    o_ref[...] = lax.rsqrt(d0_ref[...] + d1_ref[...] + 1.0)


def _prep_body(h_ref, dv_ref, w_ref, o_ref):
    o_ref[...] = jnp.dot(h_ref[...], w_ref[...],
                         preferred_element_type=jnp.float32) * dv_ref[...]


def _mid_body(a0_ref, a1_ref, tp_ref, dv_ref, b_ref, w_ref, o_ref):
    dinv = dv_ref[...]
    h = (a0_ref[...] + a1_ref[...] + tp_ref[...]) * dinv + b_ref[...]
    o_ref[...] = jnp.dot(h, w_ref[...], preferred_element_type=jnp.float32) * dinv


def _fin_body(a0_ref, a1_ref, tp_ref, dv_ref, b_ref,
              w1_ref, b1_ref, w2_ref, b2_ref, o_ref):
    h = (a0_ref[...] + a1_ref[...] + tp_ref[...]) * dv_ref[...] + b_ref[...]
    a = jnp.maximum(jnp.dot(h, w1_ref[...],
                            preferred_element_type=jnp.float32) + b1_ref[...], 0.0)
    o_ref[...] = jnp.dot(a, w2_ref[...],
                         preferred_element_type=jnp.float32) + b2_ref[...]


_GRID = (N_NODES // BR,)
_row = pl.BlockSpec((BR, H), lambda i: (i, 0))
_mat = pl.BlockSpec((H, H), lambda i: (0, 0))
_vec = pl.BlockSpec((1, H), lambda i: (0, 0))
_rowNH = jax.ShapeDtypeStruct((N_NODES, H), jnp.float32)


_encoder = pl.pallas_call(
    _enc_body, out_shape=_rowNH, grid=_GRID,
    in_specs=[_row, _vec, _vec, _mat, _vec, _mat, _vec, _vec, _vec],
    out_specs=_row)

_dinvk = pl.pallas_call(
    _dinv_body, out_shape=_rowNH, grid=_GRID,
    in_specs=[_row, _row],
    out_specs=_row)

_prep = pl.pallas_call(
    _prep_body, out_shape=_rowNH, grid=_GRID,
    in_specs=[_row, _row, _mat],
    out_specs=_row)

_mid = pl.pallas_call(
    _mid_body, out_shape=_rowNH, grid=_GRID,
    in_specs=[_row, _row, _row, _row, _vec, _mat],
    out_specs=_row)

_fin = pl.pallas_call(
    _fin_body, out_shape=jax.ShapeDtypeStruct((N_NODES, OUT), jnp.float32),
    grid=_GRID,
    in_specs=[_row, _row, _row, _row, _vec, _mat,
              _vec, pl.BlockSpec((H, OUT), lambda i: (0, 0)),
              pl.BlockSpec((1, OUT), lambda i: (0, 0))],
    out_specs=pl.BlockSpec((BR, OUT), lambda i: (i, 0)))


def kernel(x, edge_index, edge_attr, mean_x, std_x, mean_edge, std_edge,
           node_W1, node_b1, node_W2, node_b2, node_g, node_beta,
           edge_W1, edge_b1, edge_W2, edge_b2, edge_g, edge_beta,
           gcn_W, gcn_b, dec_W1, dec_b1, dec_W2, dec_b2):
    f32 = jnp.float32
    src = edge_index[0].astype(jnp.int32)
    dst = edge_index[1].astype(jnp.int32)
    order = jnp.argsort(src)
    src = src[order]
    dst = dst[order]
    e = src.shape[0]
    pad = EPAD - e
    src_p = jnp.concatenate([src, jnp.zeros((pad,), jnp.int32)])
    dst_p = jnp.concatenate([dst, jnp.full((pad,), N_NODES, jnp.int32)])
    edges_p = jnp.stack([src_p.reshape(TCH, CHUNK),
                         dst_p.reshape(TCH, CHUNK)], axis=1)
    zeros_h = jnp.zeros((NACC, H), f32)
    ones_h = jnp.ones((CHUNK, H), f32)

    row = lambda v: v.reshape(1, -1)

    edge_accum, deg_accum = _sc_kernels()

    h0 = _encoder(x, row(mean_x), row(std_x), node_W1, row(node_b1),
                  node_W2, row(node_b2), row(node_g), row(node_beta))
    deg0, deg1 = deg_accum(edges_p, ones_h, zeros_h)
    dinvb = _dinvk(deg0, deg1)
    t = _prep(h0, dinvb, gcn_W[0])
    out = None
    for l in range(L):
        acc0, acc1 = edge_accum(t, edges_p, zeros_h)
        if l < L - 1:
            t = _mid(acc0, acc1, t, dinvb, row(gcn_b[l]), gcn_W[l + 1])
        else:
            out = _fin(acc0, acc1, t, dinvb, row(gcn_b[L - 1]),
                       dec_W1, row(dec_b1), dec_W2, row(dec_b2))
    return out


# speedup vs baseline: 1.3513x; 1.3513x over previous
import functools

import jax
import jax.numpy as jnp
from jax import lax
from jax.experimental import pallas as pl
from jax.experimental.pallas import tpu as pltpu
from jax.experimental.pallas import tpu_sc as plsc

N_NODES = 10000
H = 128
OUT = 3
L = 4

NC = 2
NS = 16
NW = NC * NS
CHUNK = 128
CPW = 80
TCH = NW * CPW
CPW0 = 144
CPW1 = 16
EPAD = TCH * CHUNK
NACC = 10240
ZROWS = NACC // NS
BR = 1000


def _edge_accum_body(t_hbm, e_hbm, zeros_hbm, out0_hbm, out1_hbm,
                     eb0, eb1, rows0, rows1, shared,
                     gsem0, gsem1, ssem0, ssem1):
    c = lax.axis_index("c")
    s = lax.axis_index("s")
    base = jnp.where(c == 0, s * CPW0, 16 * CPW0 + s * CPW1)
    nch = jnp.where(c == 0, CPW0, CPW1)
    pltpu.sync_copy(zeros_hbm.at[pl.ds(s * ZROWS, ZROWS)],
                    shared.at[pl.ds(s * ZROWS, ZROWS)])
    plsc.subcore_barrier()

    eb = (eb0, eb1)
    rows = (rows0, rows1)
    gsem = (gsem0, gsem1)
    ssem = (ssem0, ssem1)

    def drain(sem):
        pltpu.make_async_copy(zeros_hbm.at[pl.ds(0, CHUNK)], rows[0],
                              sem).wait()

    pltpu.sync_copy(e_hbm.at[base], eb[0])
    pltpu.async_copy(t_hbm.at[eb[0].at[0]], rows[0], gsem[0])

    def body(j2, carry):
        for phase in range(2):
            j = 2 * j2 + phase
            cur, nxt = phase, 1 - phase

            if phase == 0:
                @pl.when(j2 > 0)
                def _():
                    drain(ssem[nxt])
                pltpu.sync_copy(e_hbm.at[base + j + 1], eb[nxt])
                pltpu.async_copy(t_hbm.at[eb[nxt].at[0]], rows[nxt],
                                 gsem[nxt])
            else:
                @pl.when(j + 1 < nch)
                def _():
                    drain(ssem[nxt])
                    pltpu.sync_copy(e_hbm.at[base + j + 1], eb[nxt])
                    pltpu.async_copy(t_hbm.at[eb[nxt].at[0]], rows[nxt],
                                     gsem[nxt])

            drain(gsem[cur])
            pltpu.async_copy(rows[cur], shared.at[eb[cur].at[1]], ssem[cur],
                             add=True)
        return carry

    lax.fori_loop(0, nch // 2, body, 0)
    drain(ssem[0])
    drain(ssem[1])
    plsc.subcore_barrier()
    @pl.when(c == 0)
    def _():
        pltpu.sync_copy(shared.at[pl.ds(s * ZROWS, ZROWS)],
                        out0_hbm.at[pl.ds(s * ZROWS, ZROWS)])
    @pl.when(c == 1)
    def _():
        pltpu.sync_copy(shared.at[pl.ds(s * ZROWS, ZROWS)],
                        out1_hbm.at[pl.ds(s * ZROWS, ZROWS)])


def _deg_accum_body(e_hbm, ones_hbm, zeros_hbm, out0_hbm, out1_hbm,
                    ev, ones_v, shared):
    c = lax.axis_index("c")
    s = lax.axis_index("s")
    w = s * NC + c
    pltpu.sync_copy(ones_hbm, ones_v)
    pltpu.sync_copy(zeros_hbm.at[pl.ds(s * ZROWS, ZROWS)],
                    shared.at[pl.ds(s * ZROWS, ZROWS)])
    plsc.subcore_barrier()

    def body(j, carry):
        pltpu.sync_copy(e_hbm.at[w * CPW + j], ev)
        pltpu.sync_copy(ones_v, shared.at[ev.at[1]], add=True)
        return carry

    lax.fori_loop(0, CPW, body, 0)
    plsc.subcore_barrier()
    @pl.when(c == 0)
    def _():
        pltpu.sync_copy(shared.at[pl.ds(s * ZROWS, ZROWS)],
                        out0_hbm.at[pl.ds(s * ZROWS, ZROWS)])
    @pl.when(c == 1)
    def _():
        pltpu.sync_copy(shared.at[pl.ds(s * ZROWS, ZROWS)],
                        out1_hbm.at[pl.ds(s * ZROWS, ZROWS)])


@functools.cache
def _sc_kernels():
    mesh = plsc.VectorSubcoreMesh(core_axis_name="c", subcore_axis_name="s")
    part = jax.ShapeDtypeStruct((NACC, H), jnp.float32)
    edge_accum = pl.kernel(
        _edge_accum_body,
        mesh=mesh,
        out_type=(part, part),
        scratch_types=[
            pltpu.VMEM((2, CHUNK), jnp.int32),
            pltpu.VMEM((2, CHUNK), jnp.int32),
            pltpu.VMEM((CHUNK, H), jnp.float32),
            pltpu.VMEM((CHUNK, H), jnp.float32),
            pltpu.VMEM_SHARED((NACC, H), jnp.float32),
            pltpu.SemaphoreType.DMA,
            pltpu.SemaphoreType.DMA,
            pltpu.SemaphoreType.DMA,
            pltpu.SemaphoreType.DMA,
        ],
    )
    deg_accum = pl.kernel(
        _deg_accum_body,
        mesh=mesh,
        out_type=(part, part),
        scratch_types=[
            pltpu.VMEM((2, CHUNK), jnp.int32),
            pltpu.VMEM((CHUNK, H), jnp.float32),
            pltpu.VMEM_SHARED((NACC, H), jnp.float32),
        ],
    )
    return edge_accum, deg_accum


def _enc_body(x_ref, mean_ref, std_ref, w1_ref, b1_ref, w2_ref, b2_ref,
              g_ref, beta_ref, o_ref):
    xn = (x_ref[...] - mean_ref[...]) / std_ref[...]
    a = jnp.maximum(jnp.dot(xn, w1_ref[...],
                            preferred_element_type=jnp.float32)
                    + b1_ref[...], 0.0)
    h = jnp.dot(a, w2_ref[...], preferred_element_type=jnp.float32) + b2_ref[...]
    mu = jnp.mean(h, axis=-1, keepdims=True)
    var = jnp.mean((h - mu) * (h - mu), axis=-1, keepdims=True)
    o_ref[...] = (h - mu) * lax.rsqrt(var + 1e-5) * g_ref[...] + beta_ref[...]


def _dinv_body(d0_ref, d1_ref, o_ref):
    o_ref[...] = lax.rsqrt(d0_ref[...] + d1_ref[...] + 1.0)


def _prep_body(h_ref, dv_ref, w_ref, o_ref):
    o_ref[...] = jnp.dot(h_ref[...], w_ref[...],
                         preferred_element_type=jnp.float32) * dv_ref[...]


def _mid_body(a0_ref, a1_ref, tp_ref, dv_ref, b_ref, w_ref, o_ref):
    dinv = dv_ref[...]
    h = (a0_ref[...] + a1_ref[...] + tp_ref[...]) * dinv + b_ref[...]
    o_ref[...] = jnp.dot(h, w_ref[...], preferred_element_type=jnp.float32) * dinv


def _fin_body(a0_ref, a1_ref, tp_ref, dv_ref, b_ref,
              w1_ref, b1_ref, w2_ref, b2_ref, o_ref):
    h = (a0_ref[...] + a1_ref[...] + tp_ref[...]) * dv_ref[...] + b_ref[...]
    a = jnp.maximum(jnp.dot(h, w1_ref[...],
                            preferred_element_type=jnp.float32) + b1_ref[...], 0.0)
    o_ref[...] = jnp.dot(a, w2_ref[...],
                         preferred_element_type=jnp.float32) + b2_ref[...]


_GRID = (N_NODES // BR,)
_row = pl.BlockSpec((BR, H), lambda i: (i, 0))
_mat = pl.BlockSpec((H, H), lambda i: (0, 0))
_vec = pl.BlockSpec((1, H), lambda i: (0, 0))
_rowNH = jax.ShapeDtypeStruct((N_NODES, H), jnp.float32)


_encoder = pl.pallas_call(
    _enc_body, out_shape=_rowNH, grid=_GRID,
    in_specs=[_row, _vec, _vec, _mat, _vec, _mat, _vec, _vec, _vec],
    out_specs=_row)

_dinvk = pl.pallas_call(
    _dinv_body, out_shape=_rowNH, grid=_GRID,
    in_specs=[_row, _row],
    out_specs=_row)

_prep = pl.pallas_call(
    _prep_body, out_shape=_rowNH, grid=_GRID,
    in_specs=[_row, _row, _mat],
    out_specs=_row)

_mid = pl.pallas_call(
    _mid_body, out_shape=_rowNH, grid=_GRID,
    in_specs=[_row, _row, _row, _row, _vec, _mat],
    out_specs=_row)

_fin = pl.pallas_call(
    _fin_body, out_shape=jax.ShapeDtypeStruct((N_NODES, OUT), jnp.float32),
    grid=_GRID,
    in_specs=[_row, _row, _row, _row, _vec, _mat,
              _vec, pl.BlockSpec((H, OUT), lambda i: (0, 0)),
              pl.BlockSpec((1, OUT), lambda i: (0, 0))],
    out_specs=pl.BlockSpec((BR, OUT), lambda i: (i, 0)))


def kernel(x, edge_index, edge_attr, mean_x, std_x, mean_edge, std_edge,
           node_W1, node_b1, node_W2, node_b2, node_g, node_beta,
           edge_W1, edge_b1, edge_W2, edge_b2, edge_g, edge_beta,
           gcn_W, gcn_b, dec_W1, dec_b1, dec_W2, dec_b2):
    f32 = jnp.float32
    src = edge_index[0].astype(jnp.int32)
    dst = edge_index[1].astype(jnp.int32)
    e = src.shape[0]
    pad = EPAD - e
    src_p = jnp.concatenate([src, jnp.zeros((pad,), jnp.int32)])
    dst_p = jnp.concatenate([dst, jnp.full((pad,), N_NODES, jnp.int32)])
    edges_p = jnp.stack([src_p.reshape(TCH, CHUNK),
                         dst_p.reshape(TCH, CHUNK)], axis=1)
    zeros_h = jnp.zeros((NACC, H), f32)
    ones_h = jnp.ones((CHUNK, H), f32)

    row = lambda v: v.reshape(1, -1)

    edge_accum, deg_accum = _sc_kernels()

    h0 = _encoder(x, row(mean_x), row(std_x), node_W1, row(node_b1),
                  node_W2, row(node_b2), row(node_g), row(node_beta))
    deg0, deg1 = deg_accum(edges_p, ones_h, zeros_h)
    dinvb = _dinvk(deg0, deg1)
    t = _prep(h0, dinvb, gcn_W[0])
    out = None
    for l in range(L):
        acc0, acc1 = edge_accum(t, edges_p, zeros_h)
        if l < L - 1:
            t = _mid(acc0, acc1, t, dinvb, row(gcn_b[l]), gcn_W[l + 1])
        else:
            out = _fin(acc0, acc1, t, dinvb, row(gcn_b[L - 1]),
                       dec_W1, row(dec_b1), dec_W2, row(dec_b2))
    return out


# speedup vs baseline: 1.3719x; 1.0152x over previous
import functools

import jax
import jax.numpy as jnp
from jax import lax
from jax.experimental import pallas as pl
from jax.experimental.pallas import tpu as pltpu
from jax.experimental.pallas import tpu_sc as plsc

N_NODES = 10000
H = 128
OUT = 3
L = 4

NC = 2
NS = 16
NW = NC * NS
CHUNK = 128
CPW = 80
TCH = NW * CPW
CPW0 = 152
CPW1 = 8
EPAD = TCH * CHUNK
NACC = 10240
ZROWS = NACC // NS
BR = 1000


def _edge_accum_body(t_hbm, e_hbm, zeros_hbm, out0_hbm, out1_hbm,
                     eb0, eb1, rows0, rows1, shared,
                     gsem0, gsem1, ssem0, ssem1):
    c = lax.axis_index("c")
    s = lax.axis_index("s")
    base = jnp.where(c == 0, s * CPW0, 16 * CPW0 + s * CPW1)
    nch = jnp.where(c == 0, CPW0, CPW1)
    pltpu.sync_copy(zeros_hbm.at[pl.ds(s * ZROWS, ZROWS)],
                    shared.at[pl.ds(s * ZROWS, ZROWS)])
    plsc.subcore_barrier()

    eb = (eb0, eb1)
    rows = (rows0, rows1)
    gsem = (gsem0, gsem1)
    ssem = (ssem0, ssem1)

    def drain(sem):
        pltpu.make_async_copy(zeros_hbm.at[pl.ds(0, CHUNK)], rows[0],
                              sem).wait()

    pltpu.sync_copy(e_hbm.at[base], eb[0])
    pltpu.async_copy(t_hbm.at[eb[0].at[0]], rows[0], gsem[0])

    def body(j2, carry):
        for phase in range(2):
            j = 2 * j2 + phase
            cur, nxt = phase, 1 - phase

            if phase == 0:
                @pl.when(j2 > 0)
                def _():
                    drain(ssem[nxt])
                pltpu.sync_copy(e_hbm.at[base + j + 1], eb[nxt])
                pltpu.async_copy(t_hbm.at[eb[nxt].at[0]], rows[nxt],
                                 gsem[nxt])
            else:
                @pl.when(j + 1 < nch)
                def _():
                    drain(ssem[nxt])
                    pltpu.sync_copy(e_hbm.at[base + j + 1], eb[nxt])
                    pltpu.async_copy(t_hbm.at[eb[nxt].at[0]], rows[nxt],
                                     gsem[nxt])

            drain(gsem[cur])
            pltpu.async_copy(rows[cur], shared.at[eb[cur].at[1]], ssem[cur],
                             add=True)
        return carry

    lax.fori_loop(0, nch // 2, body, 0)
    drain(ssem[0])
    drain(ssem[1])
    plsc.subcore_barrier()
    @pl.when(c == 0)
    def _():
        pltpu.sync_copy(shared.at[pl.ds(s * ZROWS, ZROWS)],
                        out0_hbm.at[pl.ds(s * ZROWS, ZROWS)])
    @pl.when(c == 1)
    def _():
        pltpu.sync_copy(shared.at[pl.ds(s * ZROWS, ZROWS)],
                        out1_hbm.at[pl.ds(s * ZROWS, ZROWS)])


def _deg_accum_body(e_hbm, ones_hbm, zeros_hbm, out0_hbm, out1_hbm,
                    ev, ones_v, shared):
    c = lax.axis_index("c")
    s = lax.axis_index("s")
    w = s * NC + c
    pltpu.sync_copy(ones_hbm, ones_v)
    pltpu.sync_copy(zeros_hbm.at[pl.ds(s * ZROWS, ZROWS)],
                    shared.at[pl.ds(s * ZROWS, ZROWS)])
    plsc.subcore_barrier()

    def body(j, carry):
        pltpu.sync_copy(e_hbm.at[w * CPW + j], ev)
        pltpu.sync_copy(ones_v, shared.at[ev.at[1]], add=True)
        return carry

    lax.fori_loop(0, CPW, body, 0)
    plsc.subcore_barrier()
    @pl.when(c == 0)
    def _():
        pltpu.sync_copy(shared.at[pl.ds(s * ZROWS, ZROWS)],
                        out0_hbm.at[pl.ds(s * ZROWS, ZROWS)])
    @pl.when(c == 1)
    def _():
        pltpu.sync_copy(shared.at[pl.ds(s * ZROWS, ZROWS)],
                        out1_hbm.at[pl.ds(s * ZROWS, ZROWS)])


@functools.cache
def _sc_kernels():
    mesh = plsc.VectorSubcoreMesh(core_axis_name="c", subcore_axis_name="s")
    part = jax.ShapeDtypeStruct((NACC, H), jnp.float32)
    edge_accum = pl.kernel(
        _edge_accum_body,
        mesh=mesh,
        out_type=(part, part),
        scratch_types=[
            pltpu.VMEM((2, CHUNK), jnp.int32),
            pltpu.VMEM((2, CHUNK), jnp.int32),
            pltpu.VMEM((CHUNK, H), jnp.float32),
            pltpu.VMEM((CHUNK, H), jnp.float32),
            pltpu.VMEM_SHARED((NACC, H), jnp.float32),
            pltpu.SemaphoreType.DMA,
            pltpu.SemaphoreType.DMA,
            pltpu.SemaphoreType.DMA,
            pltpu.SemaphoreType.DMA,
        ],
    )
    deg_accum = pl.kernel(
        _deg_accum_body,
        mesh=mesh,
        out_type=(part, part),
        scratch_types=[
            pltpu.VMEM((2, CHUNK), jnp.int32),
            pltpu.VMEM((CHUNK, H), jnp.float32),
            pltpu.VMEM_SHARED((NACC, H), jnp.float32),
        ],
    )
    return edge_accum, deg_accum


def _enc_body(x_ref, mean_ref, std_ref, w1_ref, b1_ref, w2_ref, b2_ref,
              g_ref, beta_ref, o_ref):
    xn = (x_ref[...] - mean_ref[...]) / std_ref[...]
    a = jnp.maximum(jnp.dot(xn, w1_ref[...],
                            preferred_element_type=jnp.float32)
                    + b1_ref[...], 0.0)
    h = jnp.dot(a, w2_ref[...], preferred_element_type=jnp.float32) + b2_ref[...]
    mu = jnp.mean(h, axis=-1, keepdims=True)
    var = jnp.mean((h - mu) * (h - mu), axis=-1, keepdims=True)
    o_ref[...] = (h - mu) * lax.rsqrt(var + 1e-5) * g_ref[...] + beta_ref[...]


def _dinv_body(d0_ref, d1_ref, o_ref):
    o_ref[...] = lax.rsqrt(d0_ref[...] + d1_ref[...] + 1.0)


def _prep_body(h_ref, dv_ref, w_ref, o_ref):
    o_ref[...] = jnp.dot(h_ref[...], w_ref[...],
                         preferred_element_type=jnp.float32) * dv_ref[...]


def _mid_body(a0_ref, a1_ref, tp_ref, dv_ref, b_ref, w_ref, o_ref):
    dinv = dv_ref[...]
    h = (a0_ref[...] + a1_ref[...] + tp_ref[...]) * dinv + b_ref[...]
    o_ref[...] = jnp.dot(h, w_ref[...], preferred_element_type=jnp.float32) * dinv


def _fin_body(a0_ref, a1_ref, tp_ref, dv_ref, b_ref,
              w1_ref, b1_ref, w2_ref, b2_ref, o_ref):
    h = (a0_ref[...] + a1_ref[...] + tp_ref[...]) * dv_ref[...] + b_ref[...]
    a = jnp.maximum(jnp.dot(h, w1_ref[...],
                            preferred_element_type=jnp.float32) + b1_ref[...], 0.0)
    o_ref[...] = jnp.dot(a, w2_ref[...],
                         preferred_element_type=jnp.float32) + b2_ref[...]


_GRID = (N_NODES // BR,)
_row = pl.BlockSpec((BR, H), lambda i: (i, 0))
_mat = pl.BlockSpec((H, H), lambda i: (0, 0))
_vec = pl.BlockSpec((1, H), lambda i: (0, 0))
_rowNH = jax.ShapeDtypeStruct((N_NODES, H), jnp.float32)


_encoder = pl.pallas_call(
    _enc_body, out_shape=_rowNH, grid=_GRID,
    in_specs=[_row, _vec, _vec, _mat, _vec, _mat, _vec, _vec, _vec],
    out_specs=_row)

_dinvk = pl.pallas_call(
    _dinv_body, out_shape=_rowNH, grid=_GRID,
    in_specs=[_row, _row],
    out_specs=_row)

_prep = pl.pallas_call(
    _prep_body, out_shape=_rowNH, grid=_GRID,
    in_specs=[_row, _row, _mat],
    out_specs=_row)

_mid = pl.pallas_call(
    _mid_body, out_shape=_rowNH, grid=_GRID,
    in_specs=[_row, _row, _row, _row, _vec, _mat],
    out_specs=_row)

_fin = pl.pallas_call(
    _fin_body, out_shape=jax.ShapeDtypeStruct((N_NODES, OUT), jnp.float32),
    grid=_GRID,
    in_specs=[_row, _row, _row, _row, _vec, _mat,
              _vec, pl.BlockSpec((H, OUT), lambda i: (0, 0)),
              pl.BlockSpec((1, OUT), lambda i: (0, 0))],
    out_specs=pl.BlockSpec((BR, OUT), lambda i: (i, 0)))


def kernel(x, edge_index, edge_attr, mean_x, std_x, mean_edge, std_edge,
           node_W1, node_b1, node_W2, node_b2, node_g, node_beta,
           edge_W1, edge_b1, edge_W2, edge_b2, edge_g, edge_beta,
           gcn_W, gcn_b, dec_W1, dec_b1, dec_W2, dec_b2):
    f32 = jnp.float32
    src = edge_index[0].astype(jnp.int32)
    dst = edge_index[1].astype(jnp.int32)
    e = src.shape[0]
    pad = EPAD - e
    src_p = jnp.concatenate([src, jnp.zeros((pad,), jnp.int32)])
    dst_p = jnp.concatenate([dst, jnp.full((pad,), N_NODES, jnp.int32)])
    edges_p = jnp.stack([src_p.reshape(TCH, CHUNK),
                         dst_p.reshape(TCH, CHUNK)], axis=1)
    zeros_h = jnp.zeros((NACC, H), f32)
    ones_h = jnp.ones((CHUNK, H), f32)

    row = lambda v: v.reshape(1, -1)

    edge_accum, deg_accum = _sc_kernels()

    h0 = _encoder(x, row(mean_x), row(std_x), node_W1, row(node_b1),
                  node_W2, row(node_b2), row(node_g), row(node_beta))
    deg0, deg1 = deg_accum(edges_p, ones_h, zeros_h)
    dinvb = _dinvk(deg0, deg1)
    t = _prep(h0, dinvb, gcn_W[0])
    out = None
    for l in range(L):
        acc0, acc1 = edge_accum(t, edges_p, zeros_h)
        if l < L - 1:
            t = _mid(acc0, acc1, t, dinvb, row(gcn_b[l]), gcn_W[l + 1])
        else:
            out = _fin(acc0, acc1, t, dinvb, row(gcn_b[L - 1]),
                       dec_W1, row(dec_b1), dec_W2, row(dec_b2))
    return out
